# baseline (speedup 1.0000x reference)
import functools

import jax
import jax.numpy as jnp
from jax import lax
from jax.experimental import pallas as pl
from jax.experimental.pallas import tpu as pltpu
from jax.experimental.pallas import tpu_sc as plsc

N = 10000
E = 320000
D = 128
N_PAD = 10240
NC = 2
NS = 16
NW = NC * NS
BLK = 128
E_PAD = 327680
EPW = E_PAD // NW
NBLK = EPW // BLK
NBLK2 = E_PAD // (NS * BLK)
RPT = N_PAD // NS
NM = 10016
RPM = NM // NS
DH = D // 2
NBUF = 5
NG = 2

_mesh = plsc.VectorSubcoreMesh(core_axis_name="c", subcore_axis_name="s")
_sc_params = pltpu.CompilerParams(needs_layout_passes=False,
                                  use_tc_tiling_on_sc=False)


@functools.partial(
    pl.kernel,
    out_type=(jax.ShapeDtypeStruct((NW, N_PAD), jnp.float32),
              jax.ShapeDtypeStruct((NW, N_PAD), jnp.float32)),
    mesh=_mesh,
    scratch_types=[
        pltpu.VMEM((NBLK, BLK), jnp.int32),
        pltpu.VMEM((NBLK, BLK), jnp.int32),
        pltpu.VMEM((N_PAD,), jnp.float32),
        pltpu.VMEM((N_PAD,), jnp.float32),
    ],
    compiler_params=_sc_params,
)
def _deg_kernel(src_hbm, dst_hbm, do_hbm, di_hbm, src_v, dst_v, acc_s, acc_d):
    c = lax.axis_index("c")
    s = lax.axis_index("s")
    wid = s * NC + c
    pltpu.sync_copy(src_hbm.at[wid], src_v)
    pltpu.sync_copy(dst_hbm.at[wid], dst_v)
    zeros16 = jnp.zeros((16,), jnp.float32)

    def zbody(i, carry):
        acc_s[pl.ds(i * 16, 16)] = zeros16
        acc_d[pl.ds(i * 16, 16)] = zeros16
        return carry

    lax.fori_loop(0, N_PAD // 16, zbody, 0)
    ones16 = jnp.ones((16,), jnp.float32)

    def jbody(j, carry):
        def kbody(k, inner):
            si = src_v[j, pl.ds(k * 16, 16)]
            plsc.addupdate_scatter(acc_s, [si], ones16)
            di = dst_v[j, pl.ds(k * 16, 16)]
            plsc.addupdate_scatter(acc_d, [di], ones16)
            return inner

        return lax.fori_loop(0, BLK // 16, kbody, carry)

    lax.fori_loop(0, NBLK, jbody, 0)
    pltpu.sync_copy(acc_s, do_hbm.at[wid])
    pltpu.sync_copy(acc_d, di_hbm.at[wid])


@functools.partial(
    pl.kernel,
    out_type=jax.ShapeDtypeStruct((NW, N_PAD), jnp.float32),
    mesh=_mesh,
    scratch_types=[
        pltpu.VMEM((NBLK, BLK), jnp.int32),
        pltpu.VMEM((NBLK, BLK), jnp.int32),
        pltpu.VMEM((N_PAD,), jnp.float32),
        pltpu.VMEM((N_PAD,), jnp.float32),
    ],
    compiler_params=_sc_params,
)
def _w_kernel(src_hbm, dst_hbm, nd_hbm, w_hbm, src_v, dst_v, ndl, wacc):
    c = lax.axis_index("c")
    s = lax.axis_index("s")
    wid = s * NC + c
    pltpu.sync_copy(src_hbm.at[wid], src_v)
    pltpu.sync_copy(dst_hbm.at[wid], dst_v)
    pltpu.sync_copy(nd_hbm, ndl)
    zeros16 = jnp.zeros((16,), jnp.float32)

    def zbody(i, carry):
        wacc[pl.ds(i * 16, 16)] = zeros16
        return carry

    lax.fori_loop(0, N_PAD // 16, zbody, 0)

    def jbody(j, carry):
        def kbody(k, inner):
            d16 = dst_v[j, pl.ds(k * 16, 16)]
            vals = plsc.load_gather(ndl, [d16])
            s16 = src_v[j, pl.ds(k * 16, 16)]
            plsc.addupdate_scatter(wacc, [s16], vals)
            return inner

        return lax.fori_loop(0, BLK // 16, kbody, carry)

    lax.fori_loop(0, NBLK, jbody, 0)
    pltpu.sync_copy(wacc, w_hbm.at[wid])


@functools.partial(
    pl.kernel,
    out_type=jax.ShapeDtypeStruct((NC, NG, N_PAD, DH), jnp.bfloat16),
    mesh=_mesh,
    scratch_types=[
        pltpu.VMEM((NBLK2, BLK), jnp.int32),
        pltpu.VMEM((NBLK2, BLK), jnp.int32),
        pltpu.VMEM((NBUF, BLK, DH), jnp.bfloat16),
        pltpu.VMEM_SHARED((NG, N_PAD, DH), jnp.bfloat16),
        pltpu.VMEM_SHARED((NM, DH), jnp.bfloat16),
    ] + [pltpu.SemaphoreType.DMA] * (2 * NBUF),
    compiler_params=_sc_params,
)
def _scatter_kernel(m0_hbm, m1_hbm, src_hbm, dst_hbm, z_hbm,
                    agg_hbm,
                    src_v, dst_v, rows_all, agg_sh, m_sh, *sems):
    c = lax.axis_index("c")
    s = lax.axis_index("s")
    rows = [rows_all.at[b] for b in range(NBUF)]
    gsem = list(sems[:NBUF])
    ssem = list(sems[NBUF:])
    m_hbm = [m0_hbm, m1_hbm]
    pltpu.sync_copy(src_hbm.at[s], src_v)
    pltpu.sync_copy(dst_hbm.at[s], dst_v)
    for ci in range(NC):
        @pl.when(c == ci)
        def _():
            pltpu.sync_copy(m_hbm[ci].at[pl.ds(s * RPT, RPT)],
                            agg_sh.at[0, pl.ds(s * RPT, RPT)])
            pltpu.sync_copy(m_hbm[ci].at[pl.ds(s * RPM, RPM)],
                            m_sh.at[pl.ds(s * RPM, RPM)])
    for gi in range(1, NG):
        pltpu.sync_copy(z_hbm, agg_sh.at[gi, pl.ds(s * RPT, RPT)])
    plsc.subcore_barrier()

    for b in range(NBUF):
        pltpu.async_copy(m_sh.at[src_v.at[b]], rows[b], gsem[b])

    def gbody(g, carry):
        for b in range(NBUF):
            j = NBUF * g + b
            pltpu.make_async_copy(m_sh.at[src_v.at[j]], rows[b],
                                  gsem[b]).wait()
            sdesc = pltpu.async_copy(rows[b],
                                     agg_sh.at[s % NG].at[dst_v.at[j]],
                                     ssem[b], add=True)
            sdesc.wait()
            @pl.when(j + NBUF < NBLK2)
            def _():
                pltpu.async_copy(m_sh.at[src_v.at[j + NBUF]],
                                 rows[b], gsem[b])
        return carry

    lax.fori_loop(0, NBLK2 // NBUF, gbody, 0)
    plsc.subcore_barrier()
    for gi in range(NG):
        pltpu.sync_copy(agg_sh.at[gi, pl.ds(s * RPT, RPT)],
                        agg_hbm.at[c, gi, pl.ds(s * RPT, RPT)])


def _prep(do_parts, di_parts):
    CB = 512

    def body(do_ref, di_ref, ns_ref, nd_ref):
        so = jnp.sum(do_ref[...], axis=0, keepdims=True) + 1.0
        si = jnp.sum(di_ref[...], axis=0, keepdims=True) + 1.0
        ns_ref[...] = lax.rsqrt(so)
        nd_ref[...] = lax.rsqrt(si)

    return pl.pallas_call(
        body,
        grid=(N_PAD // CB,),
        in_specs=[pl.BlockSpec((NW, CB), lambda i: (0, i)),
                  pl.BlockSpec((NW, CB), lambda i: (0, i))],
        out_specs=[pl.BlockSpec((1, CB), lambda i: (0, i)),
                   pl.BlockSpec((1, CB), lambda i: (0, i))],
        out_shape=[jax.ShapeDtypeStruct((1, N_PAD), jnp.float32)] * 2,
    )(do_parts, di_parts)


def _matmul(x_pad, ns_col, W1):
    RB = 256

    def body(x_ref, ns_ref, w_ref, o0_ref, o1_ref):
        m = jnp.dot(x_ref[...] * ns_ref[...], w_ref[...],
                    preferred_element_type=jnp.float32)
        o0_ref[...] = m[:, :DH].astype(jnp.bfloat16)
        o1_ref[...] = m[:, DH:].astype(jnp.bfloat16)

    return pl.pallas_call(
        body,
        grid=(N_PAD // RB,),
        in_specs=[pl.BlockSpec((RB, D), lambda i: (i, 0)),
                  pl.BlockSpec((RB, 1), lambda i: (i, 0)),
                  pl.BlockSpec((D, D), lambda i: (0, 0))],
        out_specs=[pl.BlockSpec((RB, DH), lambda i: (i, 0)),
                   pl.BlockSpec((RB, DH), lambda i: (i, 0))],
        out_shape=[jax.ShapeDtypeStruct((N_PAD, DH), jnp.bfloat16)] * 2,
    )(x_pad, ns_col, W1)


def _final(agg, nd_col, w_parts, ns_row, nd_row, b1r, W2, b2r):
    RB = 256

    def body(agg_ref, ndc_ref, wp_ref, nsr_ref, ndr_ref, b1_ref,
             w2_ref, b2_ref, out_ref, s_acc):
        i = pl.program_id(0)

        @pl.when(i == 0)
        def _():
            s_acc[...] = jnp.zeros_like(s_acc)

        halves = []
        for ci in range(NC):
            acc = agg_ref[ci, 0].astype(jnp.float32)
            for gi in range(1, NG):
                acc = acc + agg_ref[ci, gi].astype(jnp.float32)
            halves.append(acc)
        a = jnp.concatenate(halves, axis=1)
        h1 = jnp.maximum(a * ndc_ref[...] + b1_ref[...], 0.0)
        wrow = nsr_ref[...] * (jnp.sum(wp_ref[...], axis=0, keepdims=True)
                               + ndr_ref[...])
        s_acc[...] += jnp.dot(wrow, h1, preferred_element_type=jnp.float32)

        @pl.when(i == pl.num_programs(0) - 1)
        def _():
            out_ref[...] = jnp.dot(s_acc[...] * (1.0 / N), w2_ref[...],
                                   preferred_element_type=jnp.float32) + b2_ref[...]

    return pl.pallas_call(
        body,
        grid=(N_PAD // RB,),
        in_specs=[pl.BlockSpec((NC, NG, RB, DH), lambda i: (0, 0, i, 0)),
                  pl.BlockSpec((RB, 1), lambda i: (i, 0)),
                  pl.BlockSpec((NW, RB), lambda i: (0, i)),
                  pl.BlockSpec((1, RB), lambda i: (0, i)),
                  pl.BlockSpec((1, RB), lambda i: (0, i)),
                  pl.BlockSpec((1, D), lambda i: (0, 0)),
                  pl.BlockSpec((D, 2), lambda i: (0, 0)),
                  pl.BlockSpec((1, 2), lambda i: (0, 0))],
        out_specs=pl.BlockSpec((1, 2), lambda i: (0, 0)),
        out_shape=jax.ShapeDtypeStruct((1, 2), jnp.float32),
        scratch_shapes=[pltpu.VMEM((1, D), jnp.float32)],
    )(agg, nd_col, w_parts, ns_row, nd_row, b1r, W2, b2r)


def kernel(x, edge_index, W1, b1, W2, b2):
    src = edge_index[0]
    dst = edge_index[1]
    pad = E_PAD - E
    srcp = jnp.concatenate(
        [src, jnp.full((pad,), N, jnp.int32)]).reshape(NW, NBLK, BLK)
    dstp = jnp.concatenate(
        [dst, jnp.full((pad,), N, jnp.int32)]).reshape(NW, NBLK, BLK)

    srcp2 = srcp.reshape(NS, NBLK2, BLK)
    dstp2 = dstp.reshape(NS, NBLK2, BLK)

    do_parts, di_parts = _deg_kernel(srcp, dstp)
    ns_row, nd_row = _prep(do_parts, di_parts)
    ns_col = ns_row.reshape(N_PAD, 1)
    nd_1d = nd_row.reshape(N_PAD)

    x_pad = jnp.pad(x, ((0, N_PAD - N), (0, 0)))
    M0, M1 = _matmul(x_pad, ns_col, W1)

    w_parts = _w_kernel(srcp, dstp, nd_1d)
    zrows = jnp.zeros((RPT, DH), jnp.bfloat16)
    agg = _scatter_kernel(M0, M1, srcp2, dstp2, zrows)

    out = _final(agg, nd_row.reshape(N_PAD, 1), w_parts, ns_row, nd_row,
                 b1.reshape(1, D), W2, b2.reshape(1, 2))
    return out


# speedup vs baseline: 1.0055x; 1.0055x over previous
import functools

import jax
import jax.numpy as jnp
from jax import lax
from jax.experimental import pallas as pl
from jax.experimental.pallas import tpu as pltpu
from jax.experimental.pallas import tpu_sc as plsc

N = 10000
E = 320000
D = 128
N_PAD = 10240
NC = 2
NS = 16
NW = NC * NS
BLK = 128
E_PAD = 327680
EPW = E_PAD // NW
NBLK = EPW // BLK
NBLK2 = E_PAD // (NS * BLK)
RPT = N_PAD // NS
NM = 10016
RPM = NM // NS
DH = D // 2
NBUF = 4
NG = 2

_mesh = plsc.VectorSubcoreMesh(core_axis_name="c", subcore_axis_name="s")
_sc_params = pltpu.CompilerParams(needs_layout_passes=False,
                                  use_tc_tiling_on_sc=False)


@functools.partial(
    pl.kernel,
    out_type=(jax.ShapeDtypeStruct((NW, N_PAD), jnp.float32),
              jax.ShapeDtypeStruct((NW, N_PAD), jnp.float32)),
    mesh=_mesh,
    scratch_types=[
        pltpu.VMEM((NBLK, BLK), jnp.int32),
        pltpu.VMEM((NBLK, BLK), jnp.int32),
        pltpu.VMEM((N_PAD,), jnp.float32),
        pltpu.VMEM((N_PAD,), jnp.float32),
    ],
    compiler_params=_sc_params,
)
def _deg_kernel(src_hbm, dst_hbm, do_hbm, di_hbm, src_v, dst_v, acc_s, acc_d):
    c = lax.axis_index("c")
    s = lax.axis_index("s")
    wid = s * NC + c
    pltpu.sync_copy(src_hbm.at[wid], src_v)
    pltpu.sync_copy(dst_hbm.at[wid], dst_v)
    zeros16 = jnp.zeros((16,), jnp.float32)

    def zbody(i, carry):
        acc_s[pl.ds(i * 16, 16)] = zeros16
        acc_d[pl.ds(i * 16, 16)] = zeros16
        return carry

    lax.fori_loop(0, N_PAD // 16, zbody, 0)
    ones16 = jnp.ones((16,), jnp.float32)

    def jbody(j, carry):
        def kbody(k, inner):
            si = src_v[j, pl.ds(k * 16, 16)]
            plsc.addupdate_scatter(acc_s, [si], ones16)
            di = dst_v[j, pl.ds(k * 16, 16)]
            plsc.addupdate_scatter(acc_d, [di], ones16)
            return inner

        return lax.fori_loop(0, BLK // 16, kbody, carry)

    lax.fori_loop(0, NBLK, jbody, 0)
    pltpu.sync_copy(acc_s, do_hbm.at[wid])
    pltpu.sync_copy(acc_d, di_hbm.at[wid])


@functools.partial(
    pl.kernel,
    out_type=jax.ShapeDtypeStruct((NW, N_PAD), jnp.float32),
    mesh=_mesh,
    scratch_types=[
        pltpu.VMEM((NBLK, BLK), jnp.int32),
        pltpu.VMEM((NBLK, BLK), jnp.int32),
        pltpu.VMEM((N_PAD,), jnp.float32),
        pltpu.VMEM((N_PAD,), jnp.float32),
    ],
    compiler_params=_sc_params,
)
def _w_kernel(src_hbm, dst_hbm, nd_hbm, w_hbm, src_v, dst_v, ndl, wacc):
    c = lax.axis_index("c")
    s = lax.axis_index("s")
    wid = s * NC + c
    pltpu.sync_copy(src_hbm.at[wid], src_v)
    pltpu.sync_copy(dst_hbm.at[wid], dst_v)
    pltpu.sync_copy(nd_hbm, ndl)
    zeros16 = jnp.zeros((16,), jnp.float32)

    def zbody(i, carry):
        wacc[pl.ds(i * 16, 16)] = zeros16
        return carry

    lax.fori_loop(0, N_PAD // 16, zbody, 0)

    def jbody(j, carry):
        def kbody(k, inner):
            d16 = dst_v[j, pl.ds(k * 16, 16)]
            vals = plsc.load_gather(ndl, [d16])
            s16 = src_v[j, pl.ds(k * 16, 16)]
            plsc.addupdate_scatter(wacc, [s16], vals)
            return inner

        return lax.fori_loop(0, BLK // 16, kbody, carry)

    lax.fori_loop(0, NBLK, jbody, 0)
    pltpu.sync_copy(wacc, w_hbm.at[wid])


@functools.partial(
    pl.kernel,
    out_type=jax.ShapeDtypeStruct((NC, NG, N_PAD, DH), jnp.bfloat16),
    mesh=_mesh,
    scratch_types=[
        pltpu.VMEM((NBLK2, BLK), jnp.int32),
        pltpu.VMEM((NBLK2, BLK), jnp.int32),
        pltpu.VMEM((NBUF, BLK, DH), jnp.bfloat16),
        pltpu.VMEM_SHARED((NG, N_PAD, DH), jnp.bfloat16),
        pltpu.VMEM_SHARED((NM, DH), jnp.bfloat16),
    ] + [pltpu.SemaphoreType.DMA] * (2 * NBUF),
    compiler_params=_sc_params,
)
def _scatter_kernel(m0_hbm, m1_hbm, src_hbm, dst_hbm, z_hbm,
                    agg_hbm,
                    src_v, dst_v, rows_all, agg_sh, m_sh, *sems):
    c = lax.axis_index("c")
    s = lax.axis_index("s")
    rows = [rows_all.at[b] for b in range(NBUF)]
    gsem = list(sems[:NBUF])
    ssem = list(sems[NBUF:])
    m_hbm = [m0_hbm, m1_hbm]
    pltpu.sync_copy(src_hbm.at[s], src_v)
    pltpu.sync_copy(dst_hbm.at[s], dst_v)
    for ci in range(NC):
        @pl.when(c == ci)
        def _():
            pltpu.sync_copy(m_hbm[ci].at[pl.ds(s * RPT, RPT)],
                            agg_sh.at[0, pl.ds(s * RPT, RPT)])
            pltpu.sync_copy(m_hbm[ci].at[pl.ds(s * RPM, RPM)],
                            m_sh.at[pl.ds(s * RPM, RPM)])
    for gi in range(1, NG):
        pltpu.sync_copy(z_hbm, agg_sh.at[gi, pl.ds(s * RPT, RPT)])
    plsc.subcore_barrier()

    for b in range(NBUF):
        pltpu.async_copy(m_sh.at[src_v.at[b]], rows[b], gsem[b])

    def gbody(g, carry):
        for b in range(NBUF):
            j = NBUF * g + b
            pltpu.make_async_copy(m_sh.at[src_v.at[j]], rows[b],
                                  gsem[b]).wait()
            sdesc = pltpu.async_copy(rows[b],
                                     agg_sh.at[s % NG].at[dst_v.at[j]],
                                     ssem[b], add=True)
            sdesc.wait()
            @pl.when(j + NBUF < NBLK2)
            def _():
                pltpu.async_copy(m_sh.at[src_v.at[j + NBUF]],
                                 rows[b], gsem[b])
        return carry

    lax.fori_loop(0, NBLK2 // NBUF, gbody, 0)
    plsc.subcore_barrier()
    for gi in range(NG):
        pltpu.sync_copy(agg_sh.at[gi, pl.ds(s * RPT, RPT)],
                        agg_hbm.at[c, gi, pl.ds(s * RPT, RPT)])


def _col_sum(parts_blk):
    ones = jnp.ones((NW, 1), jnp.float32)
    return lax.dot_general(parts_blk, ones, (((0,), (0,)), ((), ())),
                           preferred_element_type=jnp.float32)


def _matmul(x_pad, do_parts, di_parts, W1):
    RB = 256

    def body(x_ref, do_ref, di_ref, w_ref, o0_ref, o1_ref, nd_ref):
        ns_col = lax.rsqrt(_col_sum(do_ref[...]) + 1.0)
        nd_ref[...] = lax.rsqrt(
            jnp.sum(di_ref[...], axis=0, keepdims=True) + 1.0)
        m = jnp.dot(x_ref[...] * ns_col, w_ref[...],
                    preferred_element_type=jnp.float32)
        o0_ref[...] = m[:, :DH].astype(jnp.bfloat16)
        o1_ref[...] = m[:, DH:].astype(jnp.bfloat16)

    return pl.pallas_call(
        body,
        grid=(N_PAD // RB,),
        in_specs=[pl.BlockSpec((RB, D), lambda i: (i, 0)),
                  pl.BlockSpec((NW, RB), lambda i: (0, i)),
                  pl.BlockSpec((NW, RB), lambda i: (0, i)),
                  pl.BlockSpec((D, D), lambda i: (0, 0))],
        out_specs=[pl.BlockSpec((RB, DH), lambda i: (i, 0)),
                   pl.BlockSpec((RB, DH), lambda i: (i, 0)),
                   pl.BlockSpec((1, RB), lambda i: (0, i))],
        out_shape=[jax.ShapeDtypeStruct((N_PAD, DH), jnp.bfloat16),
                   jax.ShapeDtypeStruct((N_PAD, DH), jnp.bfloat16),
                   jax.ShapeDtypeStruct((1, N_PAD), jnp.float32)],
    )(x_pad, do_parts, di_parts, W1)


def _final(agg, do_parts, di_parts, w_parts, b1r, W2, b2r):
    RB = 256

    def body(agg_ref, do_ref, di_ref, wp_ref, b1_ref,
             w2_ref, b2_ref, out_ref, s_acc):
        i = pl.program_id(0)

        @pl.when(i == 0)
        def _():
            s_acc[...] = jnp.zeros_like(s_acc)

        nd_col = lax.rsqrt(_col_sum(di_ref[...]) + 1.0)
        ns_row = lax.rsqrt(
            jnp.sum(do_ref[...], axis=0, keepdims=True) + 1.0)
        nd_row = lax.rsqrt(
            jnp.sum(di_ref[...], axis=0, keepdims=True) + 1.0)
        halves = []
        for ci in range(NC):
            acc = agg_ref[ci, 0].astype(jnp.float32)
            for gi in range(1, NG):
                acc = acc + agg_ref[ci, gi].astype(jnp.float32)
            halves.append(acc)
        a = jnp.concatenate(halves, axis=1)
        h1 = jnp.maximum(a * nd_col + b1_ref[...], 0.0)
        wrow = ns_row * (jnp.sum(wp_ref[...], axis=0, keepdims=True)
                         + nd_row)
        s_acc[...] += jnp.dot(wrow, h1, preferred_element_type=jnp.float32)

        @pl.when(i == pl.num_programs(0) - 1)
        def _():
            out_ref[...] = jnp.dot(s_acc[...] * (1.0 / N), w2_ref[...],
                                   preferred_element_type=jnp.float32) + b2_ref[...]

    return pl.pallas_call(
        body,
        grid=(N_PAD // RB,),
        in_specs=[pl.BlockSpec((NC, NG, RB, DH), lambda i: (0, 0, i, 0)),
                  pl.BlockSpec((NW, RB), lambda i: (0, i)),
                  pl.BlockSpec((NW, RB), lambda i: (0, i)),
                  pl.BlockSpec((NW, RB), lambda i: (0, i)),
                  pl.BlockSpec((1, D), lambda i: (0, 0)),
                  pl.BlockSpec((D, 2), lambda i: (0, 0)),
                  pl.BlockSpec((1, 2), lambda i: (0, 0))],
        out_specs=pl.BlockSpec((1, 2), lambda i: (0, 0)),
        out_shape=jax.ShapeDtypeStruct((1, 2), jnp.float32),
        scratch_shapes=[pltpu.VMEM((1, D), jnp.float32)],
    )(agg, do_parts, di_parts, w_parts, b1r, W2, b2r)


def kernel(x, edge_index, W1, b1, W2, b2):
    src = edge_index[0]
    dst = edge_index[1]
    pad = E_PAD - E
    srcp = jnp.concatenate(
        [src, jnp.full((pad,), N, jnp.int32)]).reshape(NW, NBLK, BLK)
    dstp = jnp.concatenate(
        [dst, jnp.full((pad,), N, jnp.int32)]).reshape(NW, NBLK, BLK)

    srcp2 = srcp.reshape(NS, NBLK2, BLK)
    dstp2 = dstp.reshape(NS, NBLK2, BLK)

    do_parts, di_parts = _deg_kernel(srcp, dstp)

    x_pad = jnp.pad(x, ((0, N_PAD - N), (0, 0)))
    M0, M1, nd_row = _matmul(x_pad, do_parts, di_parts, W1)

    w_parts = _w_kernel(srcp, dstp, nd_row.reshape(N_PAD))
    zrows = jnp.zeros((RPT, DH), jnp.bfloat16)
    agg = _scatter_kernel(M0, M1, srcp2, dstp2, zrows)

    out = _final(agg, do_parts, di_parts, w_parts,
                 b1.reshape(1, D), W2, b2.reshape(1, 2))
    return out
